# baseline (speedup 1.0000x reference)
import functools

import jax
import jax.numpy as jnp
from jax import lax
from jax.experimental import pallas as pl
from jax.experimental.pallas import tpu as pltpu
from jax.experimental.pallas import tpu_sc as plsc

DIM = 64
LANES = 16
KREGS = DIM // LANES
CHUNK = 320
NBUF = 2
EPS = 1e-6


def _allsum(v):
    idx = lax.iota(jnp.int32, LANES)
    for sh in (1, 2, 4, 8):
        v = v + v.at[idx ^ sh].get(mode="promise_in_bounds")
    return v


def _ln_body(emb_hbm, posemb_hbm, segemb_hbm, gamma_hbm, beta_hbm,
             x_hbm, seg_hbm, pos_hbm, out_hbm, *refs, tokens_per_worker):
    nc = 2
    wid = lax.axis_index("s") * nc + lax.axis_index("c")
    base0 = wid * tokens_per_worker
    nchunks = tokens_per_worker // CHUNK
    ngroups = nchunks // NBUF

    it = iter(refs)
    xidx = [[next(it) for _ in range(NBUF)] for _ in range(2)]
    pidx = [[next(it) for _ in range(NBUF)] for _ in range(2)]
    sidx = [[next(it) for _ in range(NBUF)] for _ in range(2)]
    erows = [next(it) for _ in range(NBUF)]
    outb = [next(it) for _ in range(NBUF)]
    posemb_v = next(it)
    segv = next(it)
    gb_v = next(it)
    semi = [next(it) for _ in range(NBUF)]
    semg = [next(it) for _ in range(NBUF)]
    semw = [next(it) for _ in range(NBUF)]

    pltpu.sync_copy(gamma_hbm, gb_v.at[pl.ds(0, DIM)])
    pltpu.sync_copy(beta_hbm, gb_v.at[pl.ds(DIM, DIM)])
    pltpu.sync_copy(segemb_hbm, segv)
    pltpu.sync_copy(posemb_hbm, posemb_v)
    gvec = [gb_v[pl.ds(k * LANES, LANES)] for k in range(KREGS)]
    bvec = [gb_v[pl.ds(DIM + k * LANES, LANES)] for k in range(KREGS)]
    s0 = [segv[0, pl.ds(k * LANES, LANES)] for k in range(KREGS)]
    d1 = [segv[1, pl.ds(k * LANES, LANES)] - s0[k] for k in range(KREGS)]

    def issue_idx(g, p, b):
        base = base0 + (g * NBUF + b) * CHUNK
        pltpu.async_copy(x_hbm.at[pl.ds(base, CHUNK)], xidx[p][b], semi[b])
        pltpu.async_copy(pos_hbm.at[pl.ds(base, CHUNK)], pidx[p][b], semi[b])
        pltpu.async_copy(seg_hbm.at[pl.ds(base, CHUNK)], sidx[p][b], semi[b])

    def wait_idx(p, b):
        pltpu.make_async_copy(x_hbm.at[pl.ds(0, CHUNK)], xidx[p][b], semi[b]).wait()
        pltpu.make_async_copy(pos_hbm.at[pl.ds(0, CHUNK)], pidx[p][b], semi[b]).wait()
        pltpu.make_async_copy(seg_hbm.at[pl.ds(0, CHUNK)], sidx[p][b], semi[b]).wait()

    def issue_gathers(p, b):
        pltpu.async_copy(emb_hbm.at[xidx[p][b]], erows[b], semg[b])

    def wait_gathers(p, b):
        pltpu.make_async_copy(emb_hbm.at[xidx[p][b]], erows[b], semg[b]).wait()

    def issue_scatter(g, b):
        base = base0 + (g * NBUF + b) * CHUNK
        pltpu.async_copy(outb[b], out_hbm.at[pl.ds(base, CHUNK)], semw[b])

    def wait_scatter(b):
        pltpu.make_async_copy(outb[b], out_hbm.at[pl.ds(0, CHUNK)], semw[b]).wait()

    pcols = [lax.iota(jnp.int32, LANES) + k * LANES for k in range(KREGS)]

    def compute(p, b):
        def token_body(t, tc):
            tg = (t // LANES) * LANES
            lane = jnp.full((LANES,), t - tg, jnp.int32)
            sf16 = sidx[p][b][pl.ds(tg, LANES)].astype(jnp.float32)
            spl = sf16.at[lane].get(mode="promise_in_bounds")
            pr16 = pidx[p][b][pl.ds(tg, LANES)]
            prow = pr16.at[lane].get(mode="promise_in_bounds")
            pbase = lax.shift_left(prow, 6)
            h = [erows[b][t, pl.ds(k * LANES, LANES)]
                 + plsc.load_gather(posemb_v, [pbase + pcols[k]])
                 + (s0[k] + spl * d1[k])
                 for k in range(KREGS)]
            s = (h[0] + h[1]) + (h[2] + h[3])
            mu = _allsum(s) * (1.0 / DIM)
            d = [hk - mu for hk in h]
            sq = (d[0] * d[0] + d[1] * d[1]) + (d[2] * d[2] + d[3] * d[3])
            a = _allsum(sq) * (1.0 / DIM) + EPS
            i = lax.bitcast_convert_type(a, jnp.int32)
            i = jnp.int32(0x5F3759DF) - lax.shift_right_arithmetic(i, 1)
            y = lax.bitcast_convert_type(i, jnp.float32)
            half_a = a * 0.5
            for _ in range(3):
                y = y * (1.5 - half_a * y * y)
            for k in range(KREGS):
                outb[b][t, pl.ds(k * LANES, LANES)] = d[k] * y * gvec[k] + bvec[k]
            return tc

        lax.fori_loop(0, CHUNK, token_body, 0, unroll=False)

    def step(g, p, b, first=False, idx_issue=True, gather_issue=True):
        wait_gathers(p, b)
        if not first:
            wait_scatter(b)
        compute(p, b)
        issue_scatter(g, b)
        if gather_issue:
            wait_idx(1 - p, b)
            issue_gathers(1 - p, b)
        if idx_issue:
            issue_idx(g + 2, p, b)

    for b in range(NBUF):
        issue_idx(0, 0, b)
    for b in range(NBUF):
        wait_idx(0, b)
        issue_gathers(0, b)
        issue_idx(1, 1, b)

    for b in range(NBUF):
        step(0, 0, b, first=True)
    for b in range(NBUF):
        step(1, 1, b)

    def group_pair(gg, carry):
        g = gg * 2
        for b in range(NBUF):
            step(g, 0, b)
        for b in range(NBUF):
            step(g + 1, 1, b)
        return carry

    lax.fori_loop(1, ngroups // 2 - 1, group_pair, 0, unroll=False)

    for b in range(NBUF):
        step(ngroups - 2, 0, b, idx_issue=False)
    for b in range(NBUF):
        step(ngroups - 1, 1, b, idx_issue=False, gather_issue=False)
    for b in range(NBUF):
        wait_scatter(b)


def kernel(emb, posemb, segemb, gamma, beta, x, seg, pos):
    b, l = x.shape
    n = b * l
    nw = 32
    tokens_per_worker = n // nw
    assert tokens_per_worker % (CHUNK * NBUF) == 0
    assert (tokens_per_worker // CHUNK // NBUF) % 2 == 0

    xf = x.reshape(n)
    segf = seg.reshape(n)
    posf = pos.reshape(n)
    posemb_flat = posemb.reshape(-1)

    idx_t = pltpu.VMEM((CHUNK,), jnp.int32)
    rows_t = pltpu.VMEM((CHUNK, DIM), jnp.float32)
    scratch = (
        [idx_t] * (2 * NBUF)
        + [idx_t] * (2 * NBUF)
        + [idx_t] * (2 * NBUF)
        + [rows_t] * NBUF
        + [rows_t] * NBUF
        + [pltpu.VMEM((posemb.shape[0] * DIM,), jnp.float32)]
        + [pltpu.VMEM((2, DIM), jnp.float32)]
        + [pltpu.VMEM((2 * DIM,), jnp.float32)]
        + [pltpu.SemaphoreType.DMA] * (3 * NBUF)
    )

    mesh = plsc.VectorSubcoreMesh(core_axis_name="c", subcore_axis_name="s")
    body = functools.partial(_ln_body, tokens_per_worker=tokens_per_worker)
    out = pl.kernel(
        body,
        out_type=jax.ShapeDtypeStruct((n, DIM), jnp.float32),
        mesh=mesh,
        compiler_params=pltpu.CompilerParams(use_tc_tiling_on_sc=False,
                                             needs_layout_passes=False),
        scratch_types=scratch,
    )(emb, posemb_flat, segemb, gamma, beta, xf, segf, posf)
    return out.reshape(b, l, DIM)


# speedup vs baseline: 1.3359x; 1.3359x over previous
import functools

import jax
import jax.numpy as jnp
from jax import lax
from jax.experimental import pallas as pl
from jax.experimental.pallas import tpu as pltpu
from jax.experimental.pallas import tpu_sc as plsc

DIM = 64
LANES = 16
KREGS = DIM // LANES
CHUNK = 320
NBUF = 2
EPS = 1e-6


def _allsum(v):
    idx = lax.iota(jnp.int32, LANES)
    for sh in (1, 2, 4, 8):
        v = v + v.at[idx ^ sh].get(mode="promise_in_bounds")
    return v


def _ln_body(emb_hbm, posemb_hbm, segemb_hbm, gamma_hbm, beta_hbm,
             x_hbm, seg_hbm, pos_hbm, out_hbm, *refs, tokens_per_worker):
    nc = 2
    wid = lax.axis_index("s") * nc + lax.axis_index("c")
    base0 = wid * tokens_per_worker
    nchunks = tokens_per_worker // CHUNK
    ngroups = nchunks // NBUF

    it = iter(refs)
    xidx = [[next(it) for _ in range(NBUF)] for _ in range(2)]
    pidx = [[next(it) for _ in range(NBUF)] for _ in range(2)]
    sidx = [[next(it) for _ in range(NBUF)] for _ in range(2)]
    erows = [next(it) for _ in range(NBUF)]
    outb = [next(it) for _ in range(NBUF)]
    posemb_v = next(it)
    segv = next(it)
    gb_v = next(it)
    semi = [next(it) for _ in range(NBUF)]
    semg = [next(it) for _ in range(NBUF)]
    semw = [next(it) for _ in range(NBUF)]

    pltpu.sync_copy(gamma_hbm, gb_v.at[pl.ds(0, DIM)])
    pltpu.sync_copy(beta_hbm, gb_v.at[pl.ds(DIM, DIM)])
    pltpu.sync_copy(segemb_hbm, segv)
    pltpu.sync_copy(posemb_hbm, posemb_v)
    gvec = [gb_v[pl.ds(k * LANES, LANES)] for k in range(KREGS)]
    bvec = [gb_v[pl.ds(DIM + k * LANES, LANES)] for k in range(KREGS)]
    s0 = [segv[0, pl.ds(k * LANES, LANES)] for k in range(KREGS)]
    d1 = [segv[1, pl.ds(k * LANES, LANES)] - s0[k] for k in range(KREGS)]

    def issue_idx(g, p, b):
        base = base0 + (g * NBUF + b) * CHUNK
        pltpu.async_copy(x_hbm.at[pl.ds(base, CHUNK)], xidx[p][b], semi[b])
        pltpu.async_copy(pos_hbm.at[pl.ds(base, CHUNK)], pidx[p][b], semi[b])
        pltpu.async_copy(seg_hbm.at[pl.ds(base, CHUNK)], sidx[p][b], semi[b])

    def wait_idx(p, b):
        pltpu.make_async_copy(x_hbm.at[pl.ds(0, CHUNK)], xidx[p][b], semi[b]).wait()
        pltpu.make_async_copy(pos_hbm.at[pl.ds(0, CHUNK)], pidx[p][b], semi[b]).wait()
        pltpu.make_async_copy(seg_hbm.at[pl.ds(0, CHUNK)], sidx[p][b], semi[b]).wait()

    def issue_gathers(p, b):
        pltpu.async_copy(emb_hbm.at[xidx[p][b]], erows[b], semg[b])

    def wait_gathers(p, b):
        pltpu.make_async_copy(emb_hbm.at[xidx[p][b]], erows[b], semg[b]).wait()

    def issue_scatter(g, b):
        base = base0 + (g * NBUF + b) * CHUNK
        pltpu.async_copy(outb[b], out_hbm.at[pl.ds(base, CHUNK)], semw[b])

    def wait_scatter(b):
        pltpu.make_async_copy(outb[b], out_hbm.at[pl.ds(0, CHUNK)], semw[b]).wait()

    pcols = [lax.iota(jnp.int32, LANES) + k * LANES for k in range(KREGS)]

    def compute(p, b):
        @plsc.parallel_loop(0, CHUNK, step=1, unroll=4)
        def token_body(t):
            tg = (t // LANES) * LANES
            lane = jnp.full((LANES,), t - tg, jnp.int32)
            sf16 = sidx[p][b][pl.ds(tg, LANES)].astype(jnp.float32)
            spl = sf16.at[lane].get(mode="promise_in_bounds")
            pr16 = pidx[p][b][pl.ds(tg, LANES)]
            prow = pr16.at[lane].get(mode="promise_in_bounds")
            pbase = lax.shift_left(prow, 6)
            h = [erows[b][t, pl.ds(k * LANES, LANES)]
                 + plsc.load_gather(posemb_v, [pbase + pcols[k]])
                 + (s0[k] + spl * d1[k])
                 for k in range(KREGS)]
            s = (h[0] + h[1]) + (h[2] + h[3])
            mu = _allsum(s) * (1.0 / DIM)
            d = [hk - mu for hk in h]
            sq = (d[0] * d[0] + d[1] * d[1]) + (d[2] * d[2] + d[3] * d[3])
            a = _allsum(sq) * (1.0 / DIM) + EPS
            i = lax.bitcast_convert_type(a, jnp.int32)
            i = jnp.int32(0x5F3759DF) - lax.shift_right_arithmetic(i, 1)
            y = lax.bitcast_convert_type(i, jnp.float32)
            half_a = a * 0.5
            for _ in range(3):
                y = y * (1.5 - half_a * y * y)
            for k in range(KREGS):
                outb[b][t, pl.ds(k * LANES, LANES)] = d[k] * y * gvec[k] + bvec[k]

    def step(g, p, b, first=False, idx_issue=True, gather_issue=True):
        wait_gathers(p, b)
        if not first:
            wait_scatter(b)
        compute(p, b)
        issue_scatter(g, b)
        if gather_issue:
            wait_idx(1 - p, b)
            issue_gathers(1 - p, b)
        if idx_issue:
            issue_idx(g + 2, p, b)

    for b in range(NBUF):
        issue_idx(0, 0, b)
    for b in range(NBUF):
        wait_idx(0, b)
        issue_gathers(0, b)
        issue_idx(1, 1, b)

    for b in range(NBUF):
        step(0, 0, b, first=True)
    for b in range(NBUF):
        step(1, 1, b)

    def group_pair(gg, carry):
        g = gg * 2
        for b in range(NBUF):
            step(g, 0, b)
        for b in range(NBUF):
            step(g + 1, 1, b)
        return carry

    lax.fori_loop(1, ngroups // 2 - 1, group_pair, 0, unroll=False)

    for b in range(NBUF):
        step(ngroups - 2, 0, b, idx_issue=False)
    for b in range(NBUF):
        step(ngroups - 1, 1, b, idx_issue=False, gather_issue=False)
    for b in range(NBUF):
        wait_scatter(b)


def kernel(emb, posemb, segemb, gamma, beta, x, seg, pos):
    b, l = x.shape
    n = b * l
    nw = 32
    tokens_per_worker = n // nw
    assert tokens_per_worker % (CHUNK * NBUF) == 0
    assert (tokens_per_worker // CHUNK // NBUF) % 2 == 0

    xf = x.reshape(n)
    segf = seg.reshape(n)
    posf = pos.reshape(n)
    posemb_flat = posemb.reshape(-1)

    idx_t = pltpu.VMEM((CHUNK,), jnp.int32)
    rows_t = pltpu.VMEM((CHUNK, DIM), jnp.float32)
    scratch = (
        [idx_t] * (2 * NBUF)
        + [idx_t] * (2 * NBUF)
        + [idx_t] * (2 * NBUF)
        + [rows_t] * NBUF
        + [rows_t] * NBUF
        + [pltpu.VMEM((posemb.shape[0] * DIM,), jnp.float32)]
        + [pltpu.VMEM((2, DIM), jnp.float32)]
        + [pltpu.VMEM((2 * DIM,), jnp.float32)]
        + [pltpu.SemaphoreType.DMA] * (3 * NBUF)
    )

    mesh = plsc.VectorSubcoreMesh(core_axis_name="c", subcore_axis_name="s")
    body = functools.partial(_ln_body, tokens_per_worker=tokens_per_worker)
    out = pl.kernel(
        body,
        out_type=jax.ShapeDtypeStruct((n, DIM), jnp.float32),
        mesh=mesh,
        compiler_params=pltpu.CompilerParams(use_tc_tiling_on_sc=False,
                                             needs_layout_passes=False),
        scratch_types=scratch,
    )(emb, posemb_flat, segemb, gamma, beta, xf, segf, posf)
    return out.reshape(b, l, DIM)
